# Initial kernel scaffold; baseline (speedup 1.0000x reference)
import jax
import jax.numpy as jnp
from jax.experimental import pallas as pl


def kernel(x, batch, W, b):
    raise NotImplementedError("write your pallas kernel here")


# speedup vs baseline: 1.2244x; 1.2244x over previous
import jax
import jax.numpy as jnp
from jax import lax
from jax.experimental import pallas as pl
from jax.experimental.pallas import tpu as pltpu
from jax.experimental.pallas import tpu_sc as plsc

N = 100000
C = 128
G = 512

L = 16
NCORES = 2
NSUB = 16
NW = NCORES * NSUB
RG = 4
CGRP = 8
RPW = N // RG
CHUNK = 1000
NCHUNK = RPW // CHUNK
RUNS = 520

_NEG_INF = float("-inf")


def _bcast_f32(v):
    return jnp.full((L,), 1.0, jnp.float32) * v


def _bcast_i32(v):
    return jnp.full((L,), 1, jnp.int32) * v


def _seg_partials_body(x_hbm, b_hbm, psum_hbm, pmax_hbm, pseg_hbm, pcnt_hbm,
                       xbuf, bbuf, sbuf, mbuf, segbuf, cntbuf):
    c = lax.axis_index("c")
    s = lax.axis_index("s")
    wid = c * NSUB + s
    rg = wid // CGRP
    cg = wid % CGRP
    row0 = rg * RPW
    col0 = cg * L

    zeros = jnp.zeros((L,), jnp.float32)
    neginf = jnp.full((L,), _NEG_INF, jnp.float32)
    lane_iota = lax.iota(jnp.int32, L)

    def clear_body(i, _):
        cntbuf[pl.ds(i * L, L)] = zeros
        return 0

    lax.fori_loop(0, RUNS, clear_body, 0)

    def row_body(r, carry):
        run_idx, cur_seg, cnt, sm, mx = carry
        seg = bbuf[pl.ds(r, L)][0]
        v = xbuf[r]
        is_new = seg != cur_seg

        def flush(ri):
            sbuf[pl.ds(ri * L, L)] = sm
            mbuf[pl.ds(ri * L, L)] = mx
            segbuf[pl.ds(ri * L, L)] = _bcast_i32(cur_seg)
            cntbuf[pl.ds(ri * L, L)] = _bcast_f32(cnt)
            return ri + 1

        run_idx = lax.cond(is_new, flush, lambda ri: ri, run_idx)
        new_sm = jnp.where(is_new, v, sm + v)
        new_mx = jnp.where(is_new, v, jnp.maximum(mx, v))
        new_cnt = jnp.where(is_new, 1.0, cnt + 1.0)
        return run_idx, seg, new_cnt, new_sm, new_mx

    def chunk_body(k, carry):
        r0 = row0 + k * CHUNK
        pltpu.sync_copy(x_hbm.at[pl.ds(r0, CHUNK), pl.ds(col0, L)], xbuf)
        pltpu.sync_copy(b_hbm.at[pl.ds(r0, CHUNK)], bbuf.at[pl.ds(0, CHUNK)])
        return lax.fori_loop(0, CHUNK, row_body, carry)

    carry = (jnp.int32(0), jnp.int32(-1), jnp.float32(0.0), zeros, neginf)
    run_idx, cur_seg, cnt, sm, mx = lax.fori_loop(0, NCHUNK, chunk_body, carry)
    sbuf[pl.ds(run_idx * L, L)] = sm
    mbuf[pl.ds(run_idx * L, L)] = mx
    segbuf[pl.ds(run_idx * L, L)] = _bcast_i32(cur_seg)
    cntbuf[pl.ds(run_idx * L, L)] = _bcast_f32(cnt)

    pltpu.sync_copy(sbuf, psum_hbm.at[wid])
    pltpu.sync_copy(mbuf, pmax_hbm.at[wid])
    pltpu.sync_copy(segbuf, pseg_hbm.at[wid])
    pltpu.sync_copy(cntbuf, pcnt_hbm.at[wid])


def _combine_body(psum_hbm, pmax_hbm, pseg_hbm, pcnt_hbm, mean_hbm, maxp_hbm,
                  accs, accm, accc, lsum, lmax, lseg, lcnt):
    c = lax.axis_index("c")
    s = lax.axis_index("s")
    active = jnp.logical_and(c == 0, s < CGRP)

    @pl.when(active)
    def _():
        cg = s
        zeros = jnp.zeros((L,), jnp.float32)
        neginf = jnp.full((L,), _NEG_INF, jnp.float32)

        def init_body(i, _):
            accs[pl.ds(i * L, L)] = zeros
            accm[pl.ds(i * L, L)] = neginf
            accc[pl.ds(i * L, L)] = zeros
            return 0

        lax.fori_loop(0, G, init_body, 0)

        def merge_one(i, _):
            cntv = lcnt[pl.ds(i * L, L)]

            @pl.when(cntv[0] > 0.0)
            def _():
                seg = lseg[pl.ds(i * L, L)][0]
                o = seg * L
                accs[pl.ds(o, L)] = accs[pl.ds(o, L)] + lsum[pl.ds(i * L, L)]
                accm[pl.ds(o, L)] = jnp.maximum(accm[pl.ds(o, L)],
                                                lmax[pl.ds(i * L, L)])
                accc[pl.ds(o, L)] = accc[pl.ds(o, L)] + cntv

            return 0

        for rg in range(RG):
            wid = rg * CGRP + cg
            pltpu.sync_copy(psum_hbm.at[wid], lsum)
            pltpu.sync_copy(pmax_hbm.at[wid], lmax)
            pltpu.sync_copy(pseg_hbm.at[wid], lseg)
            pltpu.sync_copy(pcnt_hbm.at[wid], lcnt)
            lax.fori_loop(0, RUNS, merge_one, 0)

        def mean_body(i, _):
            o = i * L
            inv = 1.0 / jnp.maximum(accc[pl.ds(o, L)], 1.0)
            accs[pl.ds(o, L)] = accs[pl.ds(o, L)] * inv
            return 0

        lax.fori_loop(0, G, mean_body, 0)

        pltpu.sync_copy(accs, mean_hbm.at[cg])
        pltpu.sync_copy(accm, maxp_hbm.at[cg])


def _merge_tc_body(mean_ref, maxp_ref, w_ref, b_ref, out_ref):
    mean = mean_ref[...]
    mx = maxp_ref[...]
    z = (jnp.dot(mean, w_ref[0:C, :], preferred_element_type=jnp.float32)
         + jnp.dot(mx, w_ref[C:2 * C, :], preferred_element_type=jnp.float32)
         + b_ref[...])
    alpha = jax.nn.sigmoid(z)
    out_ref[...] = alpha * mean + (1.0 - alpha) * mx


@jax.jit
def kernel(x, batch, W, b):
    batch_i = batch.astype(jnp.int32)
    f32 = jnp.float32

    seg_partials = pl.kernel(
        _seg_partials_body,
        out_type=(
            jax.ShapeDtypeStruct((NW, RUNS * L), f32),
            jax.ShapeDtypeStruct((NW, RUNS * L), f32),
            jax.ShapeDtypeStruct((NW, RUNS * L), jnp.int32),
            jax.ShapeDtypeStruct((NW, RUNS * L), f32),
        ),
        mesh=plsc.VectorSubcoreMesh(core_axis_name="c", subcore_axis_name="s"),
        compiler_params=pltpu.CompilerParams(use_tc_tiling_on_sc=False),
        scratch_types=[
            pltpu.VMEM((CHUNK, L), f32),
            pltpu.VMEM((CHUNK + L,), jnp.int32),
            pltpu.VMEM((RUNS * L,), f32),
            pltpu.VMEM((RUNS * L,), f32),
            pltpu.VMEM((RUNS * L,), jnp.int32),
            pltpu.VMEM((RUNS * L,), f32),
        ],
    )
    psum, pmax, pseg, pcnt = seg_partials(x, batch_i)

    combine = pl.kernel(
        _combine_body,
        out_type=(
            jax.ShapeDtypeStruct((CGRP, G * L), f32),
            jax.ShapeDtypeStruct((CGRP, G * L), f32),
        ),
        mesh=plsc.VectorSubcoreMesh(core_axis_name="c", subcore_axis_name="s"),
        compiler_params=pltpu.CompilerParams(use_tc_tiling_on_sc=False),
        scratch_types=[
            pltpu.VMEM((G * L,), f32),
            pltpu.VMEM((G * L,), f32),
            pltpu.VMEM((G * L,), f32),
            pltpu.VMEM((RUNS * L,), f32),
            pltpu.VMEM((RUNS * L,), f32),
            pltpu.VMEM((RUNS * L,), jnp.int32),
            pltpu.VMEM((RUNS * L,), f32),
        ],
    )
    meant, maxt = combine(psum, pmax, pseg, pcnt)

    mean = meant.reshape(CGRP, G, L).transpose(1, 0, 2).reshape(G, C)
    maxp = maxt.reshape(CGRP, G, L).transpose(1, 0, 2).reshape(G, C)

    out = pl.pallas_call(
        _merge_tc_body,
        out_shape=jax.ShapeDtypeStruct((G, C), f32),
    )(mean, maxp, W, b.reshape(1, C))
    return out


# speedup vs baseline: 2.4997x; 2.0416x over previous
import jax
import jax.numpy as jnp
from jax import lax
from jax.experimental import pallas as pl
from jax.experimental.pallas import tpu as pltpu
from jax.experimental.pallas import tpu_sc as plsc

N = 100000
C = 128
G = 512

L = 16
NCORES = 2
NSUB = 16
NW = NCORES * NSUB
RG = 4
CGRP = 8
RPW = N // RG
CHUNK = 1000
NCHUNK = RPW // CHUNK
RUNS = 520

_NEG_INF = float("-inf")


def _bcast_f32(v):
    return jnp.full((L,), 1.0, jnp.float32) * v


def _bcast_i32(v):
    return jnp.full((L,), 1, jnp.int32) * v


def _seg_partials_body(x_hbm, b_hbm, psum_hbm, pmax_hbm, pseg_hbm, pcnt_hbm,
                       xbuf, bbuf, sbuf, mbuf, segbuf, cntbuf, smbuf, mxbuf):
    c = lax.axis_index("c")
    s = lax.axis_index("s")
    wid = c * NSUB + s
    rg = wid // CGRP
    cg = wid % CGRP
    row0 = rg * RPW
    col0 = cg * L

    zeros = jnp.zeros((L,), jnp.float32)
    neginf = jnp.full((L,), _NEG_INF, jnp.float32)

    def clear_body(i, _):
        cntbuf[pl.ds(i * L, L)] = zeros
        return 0

    lax.fori_loop(0, RUNS, clear_body, 0)

    def flush_run(ri, cur_seg, cnt):
        sbuf[pl.ds(ri * L, L)] = smbuf[...]
        mbuf[pl.ds(ri * L, L)] = mxbuf[...]
        segbuf[pl.ds(ri * L, L)] = _bcast_i32(cur_seg)
        cntbuf[pl.ds(ri * L, L)] = _bcast_f32(cnt)

    def row_body(r, carry):
        run_idx, cur_seg, cnt = carry
        seg = bbuf[pl.ds(r, L)][0]
        v = xbuf[r]
        is_new = seg != cur_seg

        def start_new(args):
            ri, cs, ct = args
            flush_run(ri, cs, ct)
            smbuf[...] = v
            mxbuf[...] = v
            return ri + 1, seg, jnp.float32(1.0)

        def accum(args):
            ri, cs, ct = args
            smbuf[...] = smbuf[...] + v
            mxbuf[...] = jnp.maximum(mxbuf[...], v)
            return ri, cs, ct + 1.0

        return lax.cond(is_new, start_new, accum, carry)

    def block_body(t, carry):
        base = t * L
        bseg = bbuf[pl.ds(base, L)]
        cur_seg = carry[1]
        fast = jnp.logical_and(bseg[0] == cur_seg, bseg[L - 1] == cur_seg)

        def fast_fn(cy):
            run_idx, seg0, cnt = cy
            sm = smbuf[...]
            mx = mxbuf[...]
            for j in range(L):
                v = xbuf[base + j]
                sm = sm + v
                mx = jnp.maximum(mx, v)
            smbuf[...] = sm
            mxbuf[...] = mx
            return run_idx, seg0, cnt + float(L)

        def slow_fn(cy):
            return lax.fori_loop(base, base + L, row_body, cy)

        return lax.cond(fast, fast_fn, slow_fn, carry)

    NBLK = CHUNK // L
    TAIL = CHUNK - NBLK * L

    def chunk_body(k, carry):
        r0 = row0 + k * CHUNK
        pltpu.sync_copy(x_hbm.at[pl.ds(r0, CHUNK), pl.ds(col0, L)], xbuf)
        pltpu.sync_copy(b_hbm.at[pl.ds(r0, CHUNK)], bbuf.at[pl.ds(0, CHUNK)])
        carry = lax.fori_loop(0, NBLK, block_body, carry)
        return lax.fori_loop(NBLK * L, NBLK * L + TAIL, row_body, carry)

    smbuf[...] = zeros
    mxbuf[...] = neginf
    carry = (jnp.int32(0), jnp.int32(-1), jnp.float32(0.0))
    run_idx, cur_seg, cnt = lax.fori_loop(0, NCHUNK, chunk_body, carry)
    flush_run(run_idx, cur_seg, cnt)

    pltpu.sync_copy(sbuf, psum_hbm.at[wid])
    pltpu.sync_copy(mbuf, pmax_hbm.at[wid])
    pltpu.sync_copy(segbuf, pseg_hbm.at[wid])
    pltpu.sync_copy(cntbuf, pcnt_hbm.at[wid])


def _combine_body(psum_hbm, pmax_hbm, pseg_hbm, pcnt_hbm, mean_hbm, maxp_hbm,
                  accs, accm, accc, lsum, lmax, lseg, lcnt):
    c = lax.axis_index("c")
    s = lax.axis_index("s")
    active = jnp.logical_and(c == 0, s < CGRP)

    @pl.when(active)
    def _():
        cg = s
        zeros = jnp.zeros((L,), jnp.float32)
        neginf = jnp.full((L,), _NEG_INF, jnp.float32)

        def init_body(i, _):
            accs[pl.ds(i * L, L)] = zeros
            accm[pl.ds(i * L, L)] = neginf
            accc[pl.ds(i * L, L)] = zeros
            return 0

        lax.fori_loop(0, G, init_body, 0)

        def merge_one(i, _):
            cntv = lcnt[pl.ds(i * L, L)]

            @pl.when(cntv[0] > 0.0)
            def _():
                seg = lseg[pl.ds(i * L, L)][0]
                o = seg * L
                accs[pl.ds(o, L)] = accs[pl.ds(o, L)] + lsum[pl.ds(i * L, L)]
                accm[pl.ds(o, L)] = jnp.maximum(accm[pl.ds(o, L)],
                                                lmax[pl.ds(i * L, L)])
                accc[pl.ds(o, L)] = accc[pl.ds(o, L)] + cntv

            return 0

        for rg in range(RG):
            wid = rg * CGRP + cg
            pltpu.sync_copy(psum_hbm.at[wid], lsum)
            pltpu.sync_copy(pmax_hbm.at[wid], lmax)
            pltpu.sync_copy(pseg_hbm.at[wid], lseg)
            pltpu.sync_copy(pcnt_hbm.at[wid], lcnt)
            lax.fori_loop(0, RUNS, merge_one, 0)

        def mean_body(i, _):
            o = i * L
            inv = 1.0 / jnp.maximum(accc[pl.ds(o, L)], 1.0)
            accs[pl.ds(o, L)] = accs[pl.ds(o, L)] * inv
            return 0

        lax.fori_loop(0, G, mean_body, 0)

        pltpu.sync_copy(accs, mean_hbm.at[cg])
        pltpu.sync_copy(accm, maxp_hbm.at[cg])


def _merge_tc_body(mean_ref, maxp_ref, w_ref, b_ref, out_ref):
    mean = mean_ref[...]
    mx = maxp_ref[...]
    z = (jnp.dot(mean, w_ref[0:C, :], preferred_element_type=jnp.float32)
         + jnp.dot(mx, w_ref[C:2 * C, :], preferred_element_type=jnp.float32)
         + b_ref[...])
    alpha = jax.nn.sigmoid(z)
    out_ref[...] = alpha * mean + (1.0 - alpha) * mx


@jax.jit
def kernel(x, batch, W, b):
    batch_i = batch.astype(jnp.int32)
    f32 = jnp.float32

    seg_partials = pl.kernel(
        _seg_partials_body,
        out_type=(
            jax.ShapeDtypeStruct((NW, RUNS * L), f32),
            jax.ShapeDtypeStruct((NW, RUNS * L), f32),
            jax.ShapeDtypeStruct((NW, RUNS * L), jnp.int32),
            jax.ShapeDtypeStruct((NW, RUNS * L), f32),
        ),
        mesh=plsc.VectorSubcoreMesh(core_axis_name="c", subcore_axis_name="s"),
        compiler_params=pltpu.CompilerParams(use_tc_tiling_on_sc=False),
        scratch_types=[
            pltpu.VMEM((CHUNK, L), f32),
            pltpu.VMEM((CHUNK + L,), jnp.int32),
            pltpu.VMEM((RUNS * L,), f32),
            pltpu.VMEM((RUNS * L,), f32),
            pltpu.VMEM((RUNS * L,), jnp.int32),
            pltpu.VMEM((RUNS * L,), f32),
            pltpu.VMEM((L,), f32),
            pltpu.VMEM((L,), f32),
        ],
    )
    psum, pmax, pseg, pcnt = seg_partials(x, batch_i)

    combine = pl.kernel(
        _combine_body,
        out_type=(
            jax.ShapeDtypeStruct((CGRP, G * L), f32),
            jax.ShapeDtypeStruct((CGRP, G * L), f32),
        ),
        mesh=plsc.VectorSubcoreMesh(core_axis_name="c", subcore_axis_name="s"),
        compiler_params=pltpu.CompilerParams(use_tc_tiling_on_sc=False),
        scratch_types=[
            pltpu.VMEM((G * L,), f32),
            pltpu.VMEM((G * L,), f32),
            pltpu.VMEM((G * L,), f32),
            pltpu.VMEM((RUNS * L,), f32),
            pltpu.VMEM((RUNS * L,), f32),
            pltpu.VMEM((RUNS * L,), jnp.int32),
            pltpu.VMEM((RUNS * L,), f32),
        ],
    )
    meant, maxt = combine(psum, pmax, pseg, pcnt)

    mean = meant.reshape(CGRP, G, L).transpose(1, 0, 2).reshape(G, C)
    maxp = maxt.reshape(CGRP, G, L).transpose(1, 0, 2).reshape(G, C)

    out = pl.pallas_call(
        _merge_tc_body,
        out_shape=jax.ShapeDtypeStruct((G, C), f32),
    )(mean, maxp, W, b.reshape(1, C))
    return out


# speedup vs baseline: 2.7613x; 1.1046x over previous
import jax
import jax.numpy as jnp
from jax import lax
from jax.experimental import pallas as pl
from jax.experimental.pallas import tpu as pltpu
from jax.experimental.pallas import tpu_sc as plsc

N = 100000
C = 128
G = 512

L = 16
NCORES = 2
NSUB = 16
NW = NCORES * NSUB
RG = 4
CGRP = 8
RPW = N // RG
CHUNK = 1000
NCHUNK = RPW // CHUNK
BLK = 2 * L

_NEG_INF = float("-inf")


def _bcast_f32(v):
    return jnp.full((L,), 1.0, jnp.float32) * v


def _bcast_i32(v):
    return jnp.full((L,), 1, jnp.int32) * v


def _seg_partials_body(x_hbm, b_hbm, dsum_hbm, dmax_hbm, dcnt_hbm,
                       xbuf, bbuf, accs_d, accm_d, accc_d, smbuf, mxbuf):
    c = lax.axis_index("c")
    s = lax.axis_index("s")
    wid = c * NSUB + s
    rg = wid // CGRP
    cg = wid % CGRP
    row0 = rg * RPW
    col0 = cg * L

    zeros = jnp.zeros((L,), jnp.float32)
    neginf = jnp.full((L,), _NEG_INF, jnp.float32)

    def init_body(i, _):
        o = i * L
        accs_d[pl.ds(o, L)] = zeros
        accm_d[pl.ds(o, L)] = neginf
        accc_d[pl.ds(o, L)] = zeros
        return 0

    lax.fori_loop(0, G, init_body, 0)

    def flush_run(cur_seg, cnt):
        @pl.when(cur_seg >= 0)
        def _():
            o = cur_seg * L
            accs_d[pl.ds(o, L)] = smbuf[...]
            accm_d[pl.ds(o, L)] = mxbuf[...]
            accc_d[pl.ds(o, L)] = _bcast_f32(cnt)

    def row_body(r, carry):
        cur_seg, cnt = carry
        seg = bbuf[pl.ds(r, L)][0]
        v = xbuf[r]
        is_new = seg != cur_seg

        def start_new(args):
            cs, ct = args
            flush_run(cs, ct)
            smbuf[...] = v
            mxbuf[...] = v
            return seg, jnp.float32(1.0)

        def accum(args):
            cs, ct = args
            smbuf[...] = smbuf[...] + v
            mxbuf[...] = jnp.maximum(mxbuf[...], v)
            return cs, ct + 1.0

        return lax.cond(is_new, start_new, accum, carry)

    def block_body(t, carry):
        base = t * BLK
        cur_seg = carry[0]
        first = bbuf[pl.ds(base, L)][0]
        last = bbuf[pl.ds(base + L, L)][L - 1]
        fast = jnp.logical_and(first == cur_seg, last == cur_seg)

        def fast_fn(cy):
            seg0, cnt = cy
            sm = smbuf[...]
            mx = mxbuf[...]
            for j in range(BLK):
                v = xbuf[base + j]
                sm = sm + v
                mx = jnp.maximum(mx, v)
            smbuf[...] = sm
            mxbuf[...] = mx
            return seg0, cnt + float(BLK)

        def slow_fn(cy):
            return lax.fori_loop(base, base + BLK, row_body, cy)

        return lax.cond(fast, fast_fn, slow_fn, carry)

    NBLK = CHUNK // BLK
    TAIL = CHUNK - NBLK * BLK

    def chunk_body(k, carry):
        r0 = row0 + k * CHUNK
        pltpu.sync_copy(x_hbm.at[pl.ds(r0, CHUNK), pl.ds(col0, L)], xbuf)
        pltpu.sync_copy(b_hbm.at[pl.ds(r0, CHUNK)], bbuf.at[pl.ds(0, CHUNK)])
        carry = lax.fori_loop(0, NBLK, block_body, carry)
        return lax.fori_loop(NBLK * BLK, NBLK * BLK + TAIL, row_body, carry)

    smbuf[...] = zeros
    mxbuf[...] = neginf
    carry = (jnp.int32(-1), jnp.float32(0.0))
    cur_seg, cnt = lax.fori_loop(0, NCHUNK, chunk_body, carry)
    flush_run(cur_seg, cnt)

    pltpu.sync_copy(accs_d, dsum_hbm.at[wid])
    pltpu.sync_copy(accm_d, dmax_hbm.at[wid])
    pltpu.sync_copy(accc_d, dcnt_hbm.at[wid])


def _combine_body(dsum_hbm, dmax_hbm, dcnt_hbm, mean_hbm, maxp_hbm,
                  accs, accm, accc, lsum, lmax, lcnt):
    c = lax.axis_index("c")
    s = lax.axis_index("s")
    active = jnp.logical_and(c == 0, s < CGRP)

    @pl.when(active)
    def _():
        cg = s

        pltpu.sync_copy(dsum_hbm.at[cg], accs)
        pltpu.sync_copy(dmax_hbm.at[cg], accm)
        pltpu.sync_copy(dcnt_hbm.at[cg], accc)

        def merge_rg(rg):
            wid = rg * CGRP + cg
            pltpu.sync_copy(dsum_hbm.at[wid], lsum)
            pltpu.sync_copy(dmax_hbm.at[wid], lmax)
            pltpu.sync_copy(dcnt_hbm.at[wid], lcnt)

            def merge_one(i, _):
                o = i * L
                accs[pl.ds(o, L)] = accs[pl.ds(o, L)] + lsum[pl.ds(o, L)]
                accm[pl.ds(o, L)] = jnp.maximum(accm[pl.ds(o, L)],
                                                lmax[pl.ds(o, L)])
                accc[pl.ds(o, L)] = accc[pl.ds(o, L)] + lcnt[pl.ds(o, L)]
                return 0

            lax.fori_loop(0, G, merge_one, 0)

        for rg in range(1, RG):
            merge_rg(rg)

        def mean_body(i, _):
            o = i * L
            inv = 1.0 / jnp.maximum(accc[pl.ds(o, L)], 1.0)
            accs[pl.ds(o, L)] = accs[pl.ds(o, L)] * inv
            return 0

        lax.fori_loop(0, G, mean_body, 0)

        pltpu.sync_copy(accs, mean_hbm.at[cg])
        pltpu.sync_copy(accm, maxp_hbm.at[cg])


def _merge_tc_body(mean_ref, maxp_ref, w_ref, b_ref, out_ref):
    mean = mean_ref[...]
    mx = maxp_ref[...]
    z = (jnp.dot(mean, w_ref[0:C, :], preferred_element_type=jnp.float32)
         + jnp.dot(mx, w_ref[C:2 * C, :], preferred_element_type=jnp.float32)
         + b_ref[...])
    alpha = jax.nn.sigmoid(z)
    out_ref[...] = alpha * mean + (1.0 - alpha) * mx


@jax.jit
def kernel(x, batch, W, b):
    batch_i = batch.astype(jnp.int32)
    f32 = jnp.float32

    seg_partials = pl.kernel(
        _seg_partials_body,
        out_type=(
            jax.ShapeDtypeStruct((NW, G * L), f32),
            jax.ShapeDtypeStruct((NW, G * L), f32),
            jax.ShapeDtypeStruct((NW, G * L), f32),
        ),
        mesh=plsc.VectorSubcoreMesh(core_axis_name="c", subcore_axis_name="s"),
        compiler_params=pltpu.CompilerParams(use_tc_tiling_on_sc=False),
        scratch_types=[
            pltpu.VMEM((CHUNK, L), f32),
            pltpu.VMEM((CHUNK + L,), jnp.int32),
            pltpu.VMEM((G * L,), f32),
            pltpu.VMEM((G * L,), f32),
            pltpu.VMEM((G * L,), f32),
            pltpu.VMEM((L,), f32),
            pltpu.VMEM((L,), f32),
        ],
    )
    dsum, dmax, dcnt = seg_partials(x, batch_i)

    combine = pl.kernel(
        _combine_body,
        out_type=(
            jax.ShapeDtypeStruct((CGRP, G * L), f32),
            jax.ShapeDtypeStruct((CGRP, G * L), f32),
        ),
        mesh=plsc.VectorSubcoreMesh(core_axis_name="c", subcore_axis_name="s"),
        compiler_params=pltpu.CompilerParams(use_tc_tiling_on_sc=False),
        scratch_types=[
            pltpu.VMEM((G * L,), f32),
            pltpu.VMEM((G * L,), f32),
            pltpu.VMEM((G * L,), f32),
            pltpu.VMEM((G * L,), f32),
            pltpu.VMEM((G * L,), f32),
            pltpu.VMEM((G * L,), f32),
        ],
    )
    meant, maxt = combine(dsum, dmax, dcnt)

    mean = meant.reshape(CGRP, G, L).transpose(1, 0, 2).reshape(G, C)
    maxp = maxt.reshape(CGRP, G, L).transpose(1, 0, 2).reshape(G, C)

    out = pl.pallas_call(
        _merge_tc_body,
        out_shape=jax.ShapeDtypeStruct((G, C), f32),
    )(mean, maxp, W, b.reshape(1, C))
    return out


# speedup vs baseline: 3.7021x; 1.3407x over previous
import jax
import jax.numpy as jnp
from jax import lax
from jax.experimental import pallas as pl
from jax.experimental.pallas import tpu as pltpu
from jax.experimental.pallas import tpu_sc as plsc

N = 100000
C = 128
G = 512

L = 16
NCORES = 2
NSUB = 16
NW = NCORES * NSUB
RG = 4
CGRP = 8
RPW = N // RG
CHUNK = 1000
NCHUNK = RPW // CHUNK
BLK = 2 * L

_NEG_INF = float("-inf")


def _bcast_f32(v):
    return jnp.full((L,), 1.0, jnp.float32) * v


def _bcast_i32(v):
    return jnp.full((L,), 1, jnp.int32) * v


def _seg_partials_body(x_hbm, b_hbm, dsum_hbm, dmax_hbm, dcnt_hbm,
                       xbuf, bbuf, accs_d, accm_d, accc_d, smbuf, mxbuf):
    c = lax.axis_index("c")
    s = lax.axis_index("s")
    wid = c * NSUB + s
    rg = wid // CGRP
    cg = wid % CGRP
    row0 = rg * RPW
    col0 = cg * L

    zeros = jnp.zeros((L,), jnp.float32)
    neginf = jnp.full((L,), _NEG_INF, jnp.float32)

    def init_body(i, _):
        o = i * L
        accs_d[pl.ds(o, L)] = zeros
        accm_d[pl.ds(o, L)] = neginf
        accc_d[pl.ds(o, L)] = zeros
        return 0

    lax.fori_loop(0, G + 1, init_body, 0)

    def flush_run(cur_seg, cnt):
        o = (cur_seg + 1) * L
        accs_d[pl.ds(o, L)] = smbuf[...]
        accm_d[pl.ds(o, L)] = mxbuf[...]
        accc_d[pl.ds(o, L)] = _bcast_f32(cnt)

    def run_slow(lo, hi, carry):
        cur_seg0, cnt0 = carry
        sm0 = smbuf[...]
        mx0 = mxbuf[...]

        def rb(r, cy):
            cur_seg, cnt, sm, mx = cy
            seg = bbuf[pl.ds(r, L)][0]
            v = xbuf[r]
            o = (cur_seg + 1) * L
            accs_d[pl.ds(o, L)] = sm
            accm_d[pl.ds(o, L)] = mx
            accc_d[pl.ds(o, L)] = _bcast_f32(cnt)
            changed = seg != cur_seg
            sm = jnp.where(changed, v, sm + v)
            mx = jnp.where(changed, v, jnp.maximum(mx, v))
            cnt = jnp.where(changed, 1.0, cnt + 1.0)
            return seg, cnt, sm, mx

        cur_seg, cnt, sm, mx = lax.fori_loop(lo, hi, rb,
                                             (cur_seg0, cnt0, sm0, mx0))
        smbuf[...] = sm
        mxbuf[...] = mx
        return cur_seg, cnt

    def block_body(t, carry):
        base = t * BLK
        cur_seg = carry[0]
        first = bbuf[pl.ds(base, L)][0]
        last = bbuf[pl.ds(base + L, L)][L - 1]
        fast = jnp.logical_and(first == cur_seg, last == cur_seg)

        def fast_fn(cy):
            seg0, cnt = cy
            sm = smbuf[...]
            mx = mxbuf[...]
            for j in range(BLK):
                v = xbuf[base + j]
                sm = sm + v
                mx = jnp.maximum(mx, v)
            smbuf[...] = sm
            mxbuf[...] = mx
            return seg0, cnt + float(BLK)

        def slow_fn(cy):
            return run_slow(base, base + BLK, cy)

        return lax.cond(fast, fast_fn, slow_fn, carry)

    NBLK = CHUNK // BLK
    TAIL = CHUNK - NBLK * BLK

    def chunk_body(k, carry):
        r0 = row0 + k * CHUNK
        pltpu.sync_copy(x_hbm.at[pl.ds(r0, CHUNK), pl.ds(col0, L)], xbuf)
        pltpu.sync_copy(b_hbm.at[pl.ds(r0, CHUNK)], bbuf.at[pl.ds(0, CHUNK)])
        carry = lax.fori_loop(0, NBLK, block_body, carry)
        return run_slow(NBLK * BLK, NBLK * BLK + TAIL, carry)

    smbuf[...] = zeros
    mxbuf[...] = neginf
    carry = (jnp.int32(-1), jnp.float32(0.0))
    cur_seg, cnt = lax.fori_loop(0, NCHUNK, chunk_body, carry)
    flush_run(cur_seg, cnt)

    pltpu.sync_copy(accs_d.at[pl.ds(L, G * L)], dsum_hbm.at[wid])
    pltpu.sync_copy(accm_d.at[pl.ds(L, G * L)], dmax_hbm.at[wid])
    pltpu.sync_copy(accc_d.at[pl.ds(L, G * L)], dcnt_hbm.at[wid])


def _combine_body(dsum_hbm, dmax_hbm, dcnt_hbm, mean_hbm, maxp_hbm,
                  accs, accm, accc, lsum, lmax, lcnt):
    c = lax.axis_index("c")
    s = lax.axis_index("s")
    active = jnp.logical_and(c == 0, s < CGRP)

    @pl.when(active)
    def _():
        cg = s

        pltpu.sync_copy(dsum_hbm.at[cg], accs)
        pltpu.sync_copy(dmax_hbm.at[cg], accm)
        pltpu.sync_copy(dcnt_hbm.at[cg], accc)

        def merge_rg(rg):
            wid = rg * CGRP + cg
            pltpu.sync_copy(dsum_hbm.at[wid], lsum)
            pltpu.sync_copy(dmax_hbm.at[wid], lmax)
            pltpu.sync_copy(dcnt_hbm.at[wid], lcnt)

            def merge_one(i, _):
                o = i * L
                accs[pl.ds(o, L)] = accs[pl.ds(o, L)] + lsum[pl.ds(o, L)]
                accm[pl.ds(o, L)] = jnp.maximum(accm[pl.ds(o, L)],
                                                lmax[pl.ds(o, L)])
                accc[pl.ds(o, L)] = accc[pl.ds(o, L)] + lcnt[pl.ds(o, L)]
                return 0

            lax.fori_loop(0, G, merge_one, 0)

        for rg in range(1, RG):
            merge_rg(rg)

        def mean_body(i, _):
            o = i * L
            inv = 1.0 / jnp.maximum(accc[pl.ds(o, L)], 1.0)
            accs[pl.ds(o, L)] = accs[pl.ds(o, L)] * inv
            return 0

        lax.fori_loop(0, G, mean_body, 0)

        pltpu.sync_copy(accs, mean_hbm.at[cg])
        pltpu.sync_copy(accm, maxp_hbm.at[cg])


def _merge_tc_body(mean_ref, maxp_ref, w_ref, b_ref, out_ref):
    mean = mean_ref[...]
    mx = maxp_ref[...]
    z = (jnp.dot(mean, w_ref[0:C, :], preferred_element_type=jnp.float32)
         + jnp.dot(mx, w_ref[C:2 * C, :], preferred_element_type=jnp.float32)
         + b_ref[...])
    alpha = jax.nn.sigmoid(z)
    out_ref[...] = alpha * mean + (1.0 - alpha) * mx


@jax.jit
def kernel(x, batch, W, b):
    batch_i = batch.astype(jnp.int32)
    f32 = jnp.float32

    seg_partials = pl.kernel(
        _seg_partials_body,
        out_type=(
            jax.ShapeDtypeStruct((NW, G * L), f32),
            jax.ShapeDtypeStruct((NW, G * L), f32),
            jax.ShapeDtypeStruct((NW, G * L), f32),
        ),
        mesh=plsc.VectorSubcoreMesh(core_axis_name="c", subcore_axis_name="s"),
        compiler_params=pltpu.CompilerParams(use_tc_tiling_on_sc=False),
        scratch_types=[
            pltpu.VMEM((CHUNK, L), f32),
            pltpu.VMEM((CHUNK + L,), jnp.int32),
            pltpu.VMEM(((G + 1) * L,), f32),
            pltpu.VMEM(((G + 1) * L,), f32),
            pltpu.VMEM(((G + 1) * L,), f32),
            pltpu.VMEM((L,), f32),
            pltpu.VMEM((L,), f32),
        ],
    )
    dsum, dmax, dcnt = seg_partials(x, batch_i)

    combine = pl.kernel(
        _combine_body,
        out_type=(
            jax.ShapeDtypeStruct((CGRP, G * L), f32),
            jax.ShapeDtypeStruct((CGRP, G * L), f32),
        ),
        mesh=plsc.VectorSubcoreMesh(core_axis_name="c", subcore_axis_name="s"),
        compiler_params=pltpu.CompilerParams(use_tc_tiling_on_sc=False),
        scratch_types=[
            pltpu.VMEM((G * L,), f32),
            pltpu.VMEM((G * L,), f32),
            pltpu.VMEM((G * L,), f32),
            pltpu.VMEM((G * L,), f32),
            pltpu.VMEM((G * L,), f32),
            pltpu.VMEM((G * L,), f32),
        ],
    )
    meant, maxt = combine(dsum, dmax, dcnt)

    mean = meant.reshape(CGRP, G, L).transpose(1, 0, 2).reshape(G, C)
    maxp = maxt.reshape(CGRP, G, L).transpose(1, 0, 2).reshape(G, C)

    out = pl.pallas_call(
        _merge_tc_body,
        out_shape=jax.ShapeDtypeStruct((G, C), f32),
    )(mean, maxp, W, b.reshape(1, C))
    return out


# speedup vs baseline: 5.1533x; 1.3920x over previous
import jax
import jax.numpy as jnp
from jax import lax
from jax.experimental import pallas as pl
from jax.experimental.pallas import tpu as pltpu
from jax.experimental.pallas import tpu_sc as plsc

N = 100000
C = 128
G = 512

L = 16
NCORES = 2
NSUB = 16
NW = NCORES * NSUB
RG = 4
CGRP = 8
RPW = N // RG
CHUNK = 1000
NCHUNK = RPW // CHUNK
BLK = 2 * L

_NEG_INF = float("-inf")


def _bcast_f32(v):
    return jnp.full((L,), 1.0, jnp.float32) * v


def _seg_partials_body(x_hbm, b_hbm, dsum_hbm, dmax_hbm, dcnt_hbm,
                       xbuf0, bbuf0, xbuf1, bbuf1,
                       accs_d, accm_d, accc_d, smbuf, mxbuf, sem0, sem1):
    c = lax.axis_index("c")
    s = lax.axis_index("s")
    wid = c * NSUB + s
    rg = wid // CGRP
    cg = wid % CGRP
    row0 = rg * RPW
    col0 = cg * L

    zeros = jnp.zeros((L,), jnp.float32)
    neginf = jnp.full((L,), _NEG_INF, jnp.float32)

    def init_body(i, _):
        o = i * L
        accs_d[pl.ds(o, L)] = zeros
        accm_d[pl.ds(o, L)] = neginf
        accc_d[pl.ds(o, L)] = zeros
        return 0

    lax.fori_loop(0, G + 1, init_body, 0)

    def flush_run(cur_seg, cnt):
        o = (cur_seg + 1) * L
        accs_d[pl.ds(o, L)] = smbuf[...]
        accm_d[pl.ds(o, L)] = mxbuf[...]
        accc_d[pl.ds(o, L)] = _bcast_f32(cnt)

    NBLK = CHUNK // BLK
    TAIL = CHUNK - NBLK * BLK

    def make_chunk_processor(xbuf, bbuf):
        def run_slow(lo, hi, carry):
            cur_seg0, cnt0 = carry
            sm0 = smbuf[...]
            mx0 = mxbuf[...]

            def rb(r, cy):
                cur_seg, cnt, sm, mx = cy
                seg = bbuf[pl.ds(r, L)][0]
                v = xbuf[r]
                o = (cur_seg + 1) * L
                accs_d[pl.ds(o, L)] = sm
                accm_d[pl.ds(o, L)] = mx
                accc_d[pl.ds(o, L)] = _bcast_f32(cnt)
                changed = seg != cur_seg
                sm = jnp.where(changed, v, sm + v)
                mx = jnp.where(changed, v, jnp.maximum(mx, v))
                cnt = jnp.where(changed, 1.0, cnt + 1.0)
                return seg, cnt, sm, mx

            cur_seg, cnt, sm, mx = lax.fori_loop(lo, hi, rb,
                                                 (cur_seg0, cnt0, sm0, mx0))
            smbuf[...] = sm
            mxbuf[...] = mx
            return cur_seg, cnt

        def block_body(t, carry):
            base = t * BLK
            cur_seg = carry[0]
            first = bbuf[pl.ds(base, L)][0]
            last = bbuf[pl.ds(base + L, L)][L - 1]
            fast = jnp.logical_and(first == cur_seg, last == cur_seg)

            def fast_fn(cy):
                seg0, cnt = cy
                sm = smbuf[...]
                mx = mxbuf[...]
                for j in range(BLK):
                    v = xbuf[base + j]
                    sm = sm + v
                    mx = jnp.maximum(mx, v)
                smbuf[...] = sm
                mxbuf[...] = mx
                return seg0, cnt + float(BLK)

            def slow_fn(cy):
                return run_slow(base, base + BLK, cy)

            return lax.cond(fast, fast_fn, slow_fn, carry)

        def process(carry):
            carry = lax.fori_loop(0, NBLK, block_body, carry)
            return run_slow(NBLK * BLK, NBLK * BLK + TAIL, carry)

        return process

    proc0 = make_chunk_processor(xbuf0, bbuf0)
    proc1 = make_chunk_processor(xbuf1, bbuf1)

    def x_src(k):
        return x_hbm.at[pl.ds(row0 + k * CHUNK, CHUNK), pl.ds(col0, L)]

    def b_src(k):
        return b_hbm.at[pl.ds(row0 + k * CHUNK, CHUNK)]

    def start_dma(k, xbuf, bbuf, sem):
        pltpu.async_copy(x_src(k), xbuf, sem)
        pltpu.async_copy(b_src(k), bbuf.at[pl.ds(0, CHUNK)], sem)

    def wait_dma(k, xbuf, bbuf, sem):
        pltpu.make_async_copy(x_src(k), xbuf, sem).wait()
        pltpu.make_async_copy(b_src(k), bbuf.at[pl.ds(0, CHUNK)], sem).wait()

    smbuf[...] = zeros
    mxbuf[...] = neginf
    carry = (jnp.int32(-1), jnp.float32(0.0))

    start_dma(0, xbuf0, bbuf0, sem0)

    def pair_body(p, carry):
        k0 = 2 * p
        wait_dma(k0, xbuf0, bbuf0, sem0)
        start_dma(k0 + 1, xbuf1, bbuf1, sem1)
        carry = proc0(carry)
        wait_dma(k0 + 1, xbuf1, bbuf1, sem1)
        start_dma(k0 + 2, xbuf0, bbuf0, sem0)
        return proc1(carry)

    carry = lax.fori_loop(0, NCHUNK // 2, pair_body, carry)
    wait_dma(NCHUNK - 1, xbuf0, bbuf0, sem0)
    carry = proc0(carry)

    cur_seg, cnt = carry
    flush_run(cur_seg, cnt)

    pltpu.sync_copy(accs_d.at[pl.ds(L, G * L)], dsum_hbm.at[wid])
    pltpu.sync_copy(accm_d.at[pl.ds(L, G * L)], dmax_hbm.at[wid])
    pltpu.sync_copy(accc_d.at[pl.ds(L, G * L)], dcnt_hbm.at[wid])


def _combine_body(dsum_hbm, dmax_hbm, dcnt_hbm, mean_hbm, maxp_hbm,
                  accs, accm, accc, lsum, lmax, lcnt):
    c = lax.axis_index("c")
    s = lax.axis_index("s")
    active = jnp.logical_and(c == 0, s < CGRP)

    @pl.when(active)
    def _():
        cg = s

        pltpu.sync_copy(dsum_hbm.at[cg], accs)
        pltpu.sync_copy(dmax_hbm.at[cg], accm)
        pltpu.sync_copy(dcnt_hbm.at[cg], accc)

        def merge_rg(rg):
            wid = rg * CGRP + cg
            pltpu.sync_copy(dsum_hbm.at[wid], lsum)
            pltpu.sync_copy(dmax_hbm.at[wid], lmax)
            pltpu.sync_copy(dcnt_hbm.at[wid], lcnt)

            def merge_one(i, _):
                o = i * L
                accs[pl.ds(o, L)] = accs[pl.ds(o, L)] + lsum[pl.ds(o, L)]
                accm[pl.ds(o, L)] = jnp.maximum(accm[pl.ds(o, L)],
                                                lmax[pl.ds(o, L)])
                accc[pl.ds(o, L)] = accc[pl.ds(o, L)] + lcnt[pl.ds(o, L)]
                return 0

            lax.fori_loop(0, G, merge_one, 0)

        for rg in range(1, RG):
            merge_rg(rg)

        def mean_body(i, _):
            o = i * L
            inv = 1.0 / jnp.maximum(accc[pl.ds(o, L)], 1.0)
            accs[pl.ds(o, L)] = accs[pl.ds(o, L)] * inv
            return 0

        lax.fori_loop(0, G, mean_body, 0)

        pltpu.sync_copy(accs, mean_hbm.at[cg])
        pltpu.sync_copy(accm, maxp_hbm.at[cg])


def _merge_tc_body(mean_ref, maxp_ref, w_ref, b_ref, out_ref):
    mean = mean_ref[...]
    mx = maxp_ref[...]
    z = (jnp.dot(mean, w_ref[0:C, :], preferred_element_type=jnp.float32)
         + jnp.dot(mx, w_ref[C:2 * C, :], preferred_element_type=jnp.float32)
         + b_ref[...])
    alpha = jax.nn.sigmoid(z)
    out_ref[...] = alpha * mean + (1.0 - alpha) * mx


@jax.jit
def kernel(x, batch, W, b):
    batch_i = batch.astype(jnp.int32)
    f32 = jnp.float32

    seg_partials = pl.kernel(
        _seg_partials_body,
        out_type=(
            jax.ShapeDtypeStruct((NW, G * L), f32),
            jax.ShapeDtypeStruct((NW, G * L), f32),
            jax.ShapeDtypeStruct((NW, G * L), f32),
        ),
        mesh=plsc.VectorSubcoreMesh(core_axis_name="c", subcore_axis_name="s"),
        compiler_params=pltpu.CompilerParams(use_tc_tiling_on_sc=False),
        scratch_types=[
            pltpu.VMEM((CHUNK, L), f32),
            pltpu.VMEM((CHUNK + L,), jnp.int32),
            pltpu.VMEM((CHUNK, L), f32),
            pltpu.VMEM((CHUNK + L,), jnp.int32),
            pltpu.VMEM(((G + 1) * L,), f32),
            pltpu.VMEM(((G + 1) * L,), f32),
            pltpu.VMEM(((G + 1) * L,), f32),
            pltpu.VMEM((L,), f32),
            pltpu.VMEM((L,), f32),
            pltpu.SemaphoreType.DMA,
            pltpu.SemaphoreType.DMA,
        ],
    )
    dsum, dmax, dcnt = seg_partials(x, batch_i)

    combine = pl.kernel(
        _combine_body,
        out_type=(
            jax.ShapeDtypeStruct((CGRP, G * L), f32),
            jax.ShapeDtypeStruct((CGRP, G * L), f32),
        ),
        mesh=plsc.VectorSubcoreMesh(core_axis_name="c", subcore_axis_name="s"),
        compiler_params=pltpu.CompilerParams(use_tc_tiling_on_sc=False),
        scratch_types=[
            pltpu.VMEM((G * L,), f32),
            pltpu.VMEM((G * L,), f32),
            pltpu.VMEM((G * L,), f32),
            pltpu.VMEM((G * L,), f32),
            pltpu.VMEM((G * L,), f32),
            pltpu.VMEM((G * L,), f32),
        ],
    )
    meant, maxt = combine(dsum, dmax, dcnt)

    mean = meant.reshape(CGRP, G, L).transpose(1, 0, 2).reshape(G, C)
    maxp = maxt.reshape(CGRP, G, L).transpose(1, 0, 2).reshape(G, C)

    out = pl.pallas_call(
        _merge_tc_body,
        out_shape=jax.ShapeDtypeStruct((G, C), f32),
    )(mean, maxp, W, b.reshape(1, C))
    return out
